# baseline (speedup 1.0000x reference)
import functools

import jax
import jax.numpy as jnp
from jax import lax
from jax.experimental import pallas as pl
from jax.experimental.pallas import tpu as pltpu
from jax.experimental.pallas import tpu_sc as plsc

N = 2048
NW = 32
RPW = N // NW
ECAP = 4096
CHUNK = 64
NCHUNK = ECAP // CHUNK
NGRP = CHUNK // 16

_MESH = plsc.VectorSubcoreMesh(core_axis_name="c", subcore_axis_name="s")
_SC_PARAMS = pltpu.CompilerParams(needs_layout_passes=False)


def _wid():
    return lax.axis_index("s") * 2 + lax.axis_index("c")


def _iota():
    return lax.iota(jnp.int32, 16)


def _vgather(v, idx):
    dn = lax.GatherDimensionNumbers(
        offset_dims=(), collapsed_slice_dims=(0,), start_index_map=(0,))
    return lax.gather(v, idx[:, None], dn, slice_sizes=(1,),
                      mode=lax.GatherScatterMode.PROMISE_IN_BOUNDS)


def _prep_body(x_ref, w_ref, as_ref, an_ref, xp_ref, s_ref, n_ref, m_ref):
    xp = jnp.dot(x_ref[...], w_ref[...], preferred_element_type=jnp.float32)
    xp_ref[...] = xp
    s = jnp.dot(xp, as_ref[...], preferred_element_type=jnp.float32)
    n = jnp.dot(xp, an_ref[...], preferred_element_type=jnp.float32)
    s_ref[...] = s
    n_ref[...] = n
    e = s + jnp.max(n, axis=0, keepdims=True)
    m_ref[...] = jnp.maximum(jnp.maximum(e, 0.2 * e), 0.0)


def _prep(x, wflat, a_s, a_n):
    n_nodes = x.shape[0]
    hc = wflat.shape[1]
    h = a_s.shape[1]
    f32 = jnp.float32
    return pl.pallas_call(
        _prep_body,
        out_shape=[
            jax.ShapeDtypeStruct((n_nodes, hc), f32),
            jax.ShapeDtypeStruct((n_nodes, h), f32),
            jax.ShapeDtypeStruct((n_nodes, h), f32),
            jax.ShapeDtypeStruct((n_nodes, h), f32),
        ],
    )(x, wflat, a_s, a_n)


def _head_proj(a_vec, heads):
    return jnp.einsum('ch,hk->hck', a_vec, jnp.eye(heads, dtype=a_vec.dtype)
                      ).reshape(-1, heads)


@functools.partial(
    pl.kernel,
    out_type=[
        jax.ShapeDtypeStruct((N, 64), jnp.float32),
        jax.ShapeDtypeStruct((NW, ECAP), jnp.int32),
        jax.ShapeDtypeStruct((NW, 80), jnp.int32),
    ],
    mesh=_MESH,
    compiler_params=_SC_PARAMS,
    scratch_types=[
        pltpu.VMEM((4, N), jnp.float32),
        pltpu.VMEM((4, N), jnp.float32),
        pltpu.VMEM((ECAP,), jnp.int32),
        [pltpu.VMEM((CHUNK, 16), jnp.float32)] * 4,
        [pltpu.VMEM((CHUNK,), jnp.int32)] * 4,
        [pltpu.VMEM((CHUNK, 128), jnp.float32)] * 4,
        pltpu.VMEM((N * 8,), jnp.float32),
        pltpu.VMEM((RPW * 8,), jnp.float32),
        pltpu.VMEM((RPW * 8,), jnp.float32),
        pltpu.VMEM((RPW, 64), jnp.float32),
        pltpu.VMEM((RPW, 16), jnp.float32),
        pltpu.VMEM((RPW, 64), jnp.float32),
        pltpu.VMEM((64,), jnp.float32),
        pltpu.VMEM((80,), jnp.int32),
        pltpu.SMEM((72,), jnp.int32),
        pltpu.SemaphoreType.DMA,
        pltpu.SemaphoreType.DMA,
        [pltpu.SemaphoreType.DMA] * 4,
    ],
)
def _sc_layer1(adj, xp, sflat, nflat, mflat, bias,
               out, codes_out, rp_out,
               abuf_a, abuf_b, codebuf, wbufs, jidxs, rowss,
               nbuf, sbuf, mbuf, accb, zb, outb, bbuf, rpbuf, rp,
               sem_a, sem_b, sem_gs):
    wid = _wid()
    row0 = wid * RPW
    iota = _iota()
    lane3 = iota >> 3
    f32 = jnp.float32
    i32 = jnp.int32
    zeros16 = jnp.zeros((16,), f32)
    izeros16 = jnp.zeros((16,), i32)

    pltpu.sync_copy(sflat.at[pl.ds(wid * (RPW * 8), RPW * 8)], sbuf)
    pltpu.sync_copy(mflat.at[pl.ds(wid * (RPW * 8), RPW * 8)], mbuf)
    pltpu.sync_copy(nflat, nbuf)
    pltpu.sync_copy(bias, bbuf)

    def zloop(r, _):
        for k in range(4):
            accb[r, pl.ds(k * 16, 16)] = zeros16
        zb[r] = zeros16
        return 0

    lax.fori_loop(0, RPW, zloop, 0)

    def wpass(c, wbuf, jidx, sem, limit):
        def grp(g, _2):
            base = c * CHUNK + g * 16
            codes = codebuf[pl.ds(base, 16)]
            rv = (codes >> 11) & (RPW - 1)
            jv = codes & (N - 1)
            jidx[pl.ds(g * 16, 16)] = jv
            valid = (jnp.full((16,), base, i32) + iota) < limit
            rows16 = jnp.full((16,), g * 16, i32) + iota
            for h in range(8):
                sh = plsc.load_gather(sbuf, [rv * 8 + h])
                mh = plsc.load_gather(mbuf, [rv * 8 + h])
                nh = plsc.load_gather(nbuf, [jv * 8 + h])
                e = sh + nh
                e = jnp.maximum(e, 0.2 * e)
                w = jnp.exp(e - mh)
                w = jnp.where(valid, w, 0.0)
                plsc.store_scatter(
                    wbuf, [rows16, jnp.full((16,), h, i32)], w)
            return 0

        lax.fori_loop(0, NGRP, grp, 0)

    def rowproc(c, wbuf, jidx, rows_v, sem, limit):
        pltpu.make_async_copy(xp.at[jidx], rows_v, sem).wait()

        cfirst = codebuf[pl.ds(c * CHUNK, 16)]
        rlo = jnp.sum(jnp.where(iota == 0, (cfirst >> 11) & (RPW - 1), 0))
        last = jnp.minimum((c + 1) * CHUNK, limit) - 1
        clast = codebuf[pl.ds((last >> 4) << 4, 16)]
        rhi = jnp.sum(jnp.where(iota == (last & 15),
                                (clast >> 11) & (RPW - 1), 0))

        def row(r, _2):
            lo = jnp.maximum(rp[r] - c * CHUNK, 0)
            hi = jnp.minimum(rp[r + 1] - c * CHUNK, CHUNK)

            @pl.when(hi > lo)
            def _():
                def edge(t, carry):
                    a0, a1, a2, a3, z = carry
                    wrow = wbuf[t]
                    z = z + wrow
                    accs = []
                    for k, ak in enumerate((a0, a1, a2, a3)):
                        xv = rows_v[t, pl.ds(k * 16, 16)]
                        wk = _vgather(wrow, 2 * k + lane3)
                        accs.append(ak + xv * wk)
                    return (*accs, z)

                init = (zeros16,) * 5
                a0, a1, a2, a3, z = lax.fori_loop(lo, hi, edge, init)
                for k, ak in enumerate((a0, a1, a2, a3)):
                    plsc.addupdate(accb.at[r, pl.ds(k * 16, 16)], ak)
                plsc.addupdate(zb.at[r], z)
            return 0

        lax.fori_loop(rlo, rhi + 1, row, 0)

    def fire1(c, limit):
        for u in range(4):
            @pl.when((c & 3) == u)
            def _(u=u):
                wpass(c, wbufs[u], jidxs[u], sem_gs[u], limit)
                pltpu.make_async_copy(
                    xp.at[jidxs[u]], rowss[u], sem_gs[u]).start()

    def proc1(c, limit):
        for u in range(4):
            @pl.when((c & 3) == u)
            def _(u=u):
                rowproc(c, wbufs[u], jidxs[u], rowss[u], sem_gs[u], limit)

    rp[0] = 0
    pltpu.make_async_copy(adj.at[pl.ds(row0, 4)], abuf_a, sem_a).start()

    def block(bi, ptr, rpvec, cf, buf, sem, nxt_buf, nxt_sem):
        pltpu.make_async_copy(
            adj.at[pl.ds(row0 + bi * 4, 4)], buf, sem).wait()

        @pl.when(bi + 1 < 16)
        def _():
            pltpu.make_async_copy(
                adj.at[pl.ds(row0 + (bi + 1) * 4, 4)],
                nxt_buf, nxt_sem).start()

        for rr in range(4):
            r = bi * 4 + rr

            def quad(k4, p):
                for u in range(4):
                    k = k4 * 4 + u
                    av = buf[rr, pl.ds(k * 16, 16)]
                    msk = av > 0.0
                    cntv = plsc.all_reduce_population_count(msk)
                    cvec = jnp.full((16,), r * N + k * 16, i32) + iota
                    pc = jnp.minimum(p, ECAP - 16)
                    plsc.store_compressed(codebuf.at[pl.ds(pc, 16)], cvec,
                                          mask=msk)
                    p = p + cntv[0]
                return p

            ptr = lax.fori_loop(0, N // 64, quad, ptr)
            pmin = jnp.minimum(ptr, ECAP)
            rp[r + 1] = pmin
            iloc = (r + 1) & 15
            rpvec = jnp.where(iota == iloc, jnp.full((16,), pmin, i32),
                              rpvec)

            @pl.when(iloc == 15)
            def _():
                rpbuf[pl.ds(r + 1 - 15, 16)] = rpvec

            rpvec = jnp.where(jnp.full((16,), iloc == 15, jnp.bool_),
                              izeros16, rpvec)

        pmin = jnp.minimum(ptr, ECAP)

        def fbody(c):
            fire1(c, pmin)

            @pl.when(c >= 3)
            def _():
                proc1(c - 3, pmin)
            return c + 1

        cf = lax.while_loop(lambda c: (c + 1) * CHUNK <= pmin, fbody, cf)
        return ptr, rpvec, cf

    def block2(ob, carry):
        ptr, rpvec, cf = carry
        ptr, rpvec, cf = block(2 * ob, ptr, rpvec, cf,
                               abuf_a, sem_a, abuf_b, sem_b)
        ptr, rpvec, cf = block(2 * ob + 1, ptr, rpvec, cf,
                               abuf_b, sem_b, abuf_a, sem_a)
        return ptr, rpvec, cf

    total, rpvec, cf = lax.fori_loop(0, 8, block2, (0, izeros16, 0))
    total = jnp.minimum(total, ECAP)
    rpbuf[pl.ds(64, 16)] = rpvec

    pltpu.sync_copy(codebuf, codes_out.at[wid])
    pltpu.sync_copy(rpbuf, rp_out.at[wid])

    nact = (total + CHUNK - 1) // CHUNK

    def dfire(c, _):
        fire1(c, total)

        @pl.when(c >= 3)
        def _():
            proc1(c - 3, total)
        return 0

    lax.fori_loop(cf, nact, dfire, 0)
    lax.fori_loop(jnp.maximum(nact - 3, 0), nact,
                  lambda c, _: (proc1(c, total), 0)[1], 0)

    def fin(r, _):
        zrow = zb[r]
        for k in range(4):
            zk = _vgather(zrow, 2 * k + lane3)
            t = accb[r, pl.ds(k * 16, 16)] / zk + bbuf[pl.ds(k * 16, 16)]
            outb[r, pl.ds(k * 16, 16)] = jnp.where(t > 0.0, t,
                                                   jnp.exp(t) - 1.0)
        return 0

    lax.fori_loop(0, RPW, fin, 0)
    pltpu.sync_copy(outb, out.at[pl.ds(row0, RPW)])


@functools.partial(
    pl.kernel,
    out_type=jax.ShapeDtypeStruct((N, 16), jnp.float32),
    mesh=_MESH,
    compiler_params=_SC_PARAMS,
    scratch_types=[
        pltpu.VMEM((ECAP,), jnp.int32),
        pltpu.VMEM((N * 16,), jnp.float32),
        pltpu.VMEM((N,), jnp.float32),
        pltpu.VMEM((RPW,), jnp.float32),
        pltpu.VMEM((RPW,), jnp.float32),
        pltpu.VMEM((ECAP,), jnp.float32),
        pltpu.VMEM((RPW, 16), jnp.float32),
        pltpu.VMEM((RPW, 16), jnp.float32),
        pltpu.VMEM((RPW, 16), jnp.float32),
        pltpu.VMEM((16,), jnp.float32),
        pltpu.VMEM((80,), jnp.int32),
        pltpu.SMEM((72,), jnp.int32),
    ],
)
def _sc_layer2(xp2flat, s2, n2, m2, bias2, codes_in, rp_in,
               out,
               codebuf, xbuf, nbuf, sbuf, mbuf, wb,
               accb, zb, outb, bbuf, rpbuf, rp):
    wid = _wid()
    row0 = wid * RPW
    iota = _iota()
    f32 = jnp.float32
    i32 = jnp.int32
    zeros16 = jnp.zeros((16,), f32)

    pltpu.sync_copy(codes_in.at[wid], codebuf)
    pltpu.sync_copy(rp_in.at[wid], rpbuf)
    pltpu.sync_copy(xp2flat, xbuf)
    pltpu.sync_copy(n2, nbuf)
    pltpu.sync_copy(s2.at[pl.ds(row0, RPW)], sbuf)
    pltpu.sync_copy(m2.at[pl.ds(row0, RPW)], mbuf)
    pltpu.sync_copy(bias2, bbuf)

    def rpl(i, _):
        vec = rpbuf[pl.ds((i >> 4) << 4, 16)]
        rp[i] = jnp.sum(jnp.where(iota == (i & 15), vec, 0))
        return 0

    lax.fori_loop(0, RPW + 1, rpl, 0)
    total = rp[RPW]

    def grp(g, _):
        base = g * 16
        codes = codebuf[pl.ds(base, 16)]
        rv = (codes >> 11) & (RPW - 1)
        jv = codes & (N - 1)
        valid = (jnp.full((16,), base, i32) + iota) < total
        sh = plsc.load_gather(sbuf, [rv])
        mh = plsc.load_gather(mbuf, [rv])
        nh = plsc.load_gather(nbuf, [jv])
        e = sh + nh
        e = jnp.maximum(e, 0.2 * e)
        w = jnp.exp(e - mh)
        wb[pl.ds(base, 16)] = jnp.where(valid, w, 0.0)
        return 0

    ngrp_used = (total + 15) >> 4
    lax.fori_loop(0, ngrp_used, grp, 0)

    def row(r, _):
        lo = rp[r]
        hi = rp[r + 1]

        @pl.when(hi > lo)
        def _():
            def edge(t, carry):
                av, z = carry
                tb = (t >> 4) << 4
                tl = t & 15
                w16 = wb[pl.ds(tb, 16)]
                wsp = _vgather(w16, jnp.full((16,), tl, i32))
                jvec = codebuf[pl.ds(tb, 16)]
                j = jnp.sum(jnp.where(iota == tl, jvec & (N - 1), 0))
                xv = xbuf[pl.ds(j * 16, 16)]
                return (av + xv * wsp, z + wsp)

            av, z = lax.fori_loop(lo, hi, edge, (zeros16, zeros16))
            accb[r] = av
            zb[r] = z
        return 0

    def zloop(r, _):
        accb[r] = zeros16
        zb[r] = jnp.full((16,), 1.0, f32)
        return 0

    lax.fori_loop(0, RPW, zloop, 0)
    lax.fori_loop(0, RPW, row, 0)

    neg = jnp.full((16,), -1e30, f32)

    def fin(r, _):
        t = accb[r] / zb[r] + bbuf[...]
        t = jnp.where(iota < 3, t, neg)
        ex = jnp.exp(t - jnp.max(t))
        outb[r] = ex / jnp.sum(ex)
        return 0

    lax.fori_loop(0, RPW, fin, 0)
    pltpu.sync_copy(outb, out.at[pl.ds(row0, RPW)])


def kernel(x, fltr, W1, a1_self, a1_neigh, b1, W2, a2_self, a2_neigh, b2):
    f, h1, c1 = W1.shape
    hc1, h2, c2 = W2.shape
    f32 = jnp.float32

    w1pad = jnp.zeros((f, 128), f32).at[:, :h1 * c1].set(
        W1.reshape(f, h1 * c1))
    a1s = jnp.zeros((128, h1), f32).at[:h1 * c1].set(_head_proj(a1_self, h1))
    a1n = jnp.zeros((128, h1), f32).at[:h1 * c1].set(_head_proj(a1_neigh, h1))
    xp1, s1, n1, m1 = _prep(x, w1pad, a1s, a1n)

    hid, codes, rps = _sc_layer1(
        fltr, xp1, s1.reshape(-1), n1.reshape(-1), m1.reshape(-1), b1)

    w2pad = jnp.zeros((hc1, 16), f32).at[:, :c2].set(W2.reshape(hc1, h2 * c2))
    a2s = jnp.zeros((16, 1), f32).at[:c2].set(_head_proj(a2_self, h2))
    a2n = jnp.zeros((16, 1), f32).at[:c2].set(_head_proj(a2_neigh, h2))
    b2pad = jnp.zeros((16,), f32).at[:c2].set(b2)

    xp2, s2, n2, m2 = _prep(hid, w2pad, a2s, a2n)
    out = _sc_layer2(xp2.reshape(-1), s2.reshape(-1), n2.reshape(-1),
                     m2.reshape(-1), b2pad, codes, rps)
    return out[:, :c2]


# speedup vs baseline: 1.4126x; 1.4126x over previous
import functools

import jax
import jax.numpy as jnp
from jax import lax
from jax.experimental import pallas as pl
from jax.experimental.pallas import tpu as pltpu
from jax.experimental.pallas import tpu_sc as plsc

N = 2048
SCROWS = 1024
NW = 32
RPW = SCROWS // NW
NBLK = RPW // 4
ECAP = 4096
CHUNK = 64
NCHUNK = ECAP // CHUNK
NGRP = CHUNK // 16

_MESH = plsc.VectorSubcoreMesh(core_axis_name="c", subcore_axis_name="s")
_SC_PARAMS = pltpu.CompilerParams(needs_layout_passes=False)


def _wid():
    return lax.axis_index("s") * 2 + lax.axis_index("c")


def _iota():
    return lax.iota(jnp.int32, 16)


def _vgather(v, idx):
    dn = lax.GatherDimensionNumbers(
        offset_dims=(), collapsed_slice_dims=(0,), start_index_map=(0,))
    return lax.gather(v, idx[:, None], dn, slice_sizes=(1,),
                      mode=lax.GatherScatterMode.PROMISE_IN_BOUNDS)


def _prep_body(x_ref, w_ref, as_ref, an_ref, xp_ref, s_ref, n_ref, m_ref):
    xp = jnp.dot(x_ref[...], w_ref[...], preferred_element_type=jnp.float32)
    xp_ref[...] = xp
    s = jnp.dot(xp, as_ref[...], preferred_element_type=jnp.float32)
    n = jnp.dot(xp, an_ref[...], preferred_element_type=jnp.float32)
    s_ref[...] = s
    n_ref[...] = n
    e = s + jnp.max(n, axis=0, keepdims=True)
    m_ref[...] = jnp.maximum(jnp.maximum(e, 0.2 * e), 0.0)


def _prep(x, wflat, a_s, a_n):
    n_nodes = x.shape[0]
    hc = wflat.shape[1]
    h = a_s.shape[1]
    f32 = jnp.float32
    return pl.pallas_call(
        _prep_body,
        out_shape=[
            jax.ShapeDtypeStruct((n_nodes, hc), f32),
            jax.ShapeDtypeStruct((n_nodes, h), f32),
            jax.ShapeDtypeStruct((n_nodes, h), f32),
            jax.ShapeDtypeStruct((n_nodes, h), f32),
        ],
    )(x, wflat, a_s, a_n)


def _head_proj(a_vec, heads):
    return jnp.einsum('ch,hk->hck', a_vec, jnp.eye(heads, dtype=a_vec.dtype)
                      ).reshape(-1, heads)


@functools.partial(
    pl.kernel,
    out_type=[
        jax.ShapeDtypeStruct((SCROWS, 64), jnp.float32),
        jax.ShapeDtypeStruct((NW, ECAP), jnp.int32),
        jax.ShapeDtypeStruct((NW, 80), jnp.int32),
    ],
    mesh=_MESH,
    compiler_params=_SC_PARAMS,
    scratch_types=[
        pltpu.VMEM((4, N), jnp.float32),
        pltpu.VMEM((4, N), jnp.float32),
        pltpu.VMEM((ECAP,), jnp.int32),
        [pltpu.VMEM((CHUNK, 16), jnp.float32)] * 4,
        [pltpu.VMEM((CHUNK,), jnp.int32)] * 4,
        [pltpu.VMEM((CHUNK, 128), jnp.float32)] * 4,
        pltpu.VMEM((N * 8,), jnp.float32),
        pltpu.VMEM((RPW * 8,), jnp.float32),
        pltpu.VMEM((RPW * 8,), jnp.float32),
        pltpu.VMEM((RPW, 64), jnp.float32),
        pltpu.VMEM((RPW, 16), jnp.float32),
        pltpu.VMEM((RPW, 64), jnp.float32),
        pltpu.VMEM((64,), jnp.float32),
        pltpu.VMEM((80,), jnp.int32),
        pltpu.SMEM((72,), jnp.int32),
        pltpu.SemaphoreType.DMA,
        pltpu.SemaphoreType.DMA,
        [pltpu.SemaphoreType.DMA] * 4,
    ],
)
def _sc_layer1(adj, xp, sflat, nflat, mflat, bias,
               out, codes_out, rp_out,
               abuf_a, abuf_b, codebuf, wbufs, jidxs, rowss,
               nbuf, sbuf, mbuf, accb, zb, outb, bbuf, rpbuf, rp,
               sem_a, sem_b, sem_gs):
    wid = _wid()
    row0 = wid * RPW
    iota = _iota()
    lane3 = iota >> 3
    f32 = jnp.float32
    i32 = jnp.int32
    zeros16 = jnp.zeros((16,), f32)
    izeros16 = jnp.zeros((16,), i32)

    pltpu.sync_copy(sflat.at[pl.ds(wid * (RPW * 8), RPW * 8)], sbuf)
    pltpu.sync_copy(mflat.at[pl.ds(wid * (RPW * 8), RPW * 8)], mbuf)
    pltpu.sync_copy(nflat, nbuf)
    pltpu.sync_copy(bias, bbuf)

    def zloop(r, _):
        for k in range(4):
            accb[r, pl.ds(k * 16, 16)] = zeros16
        zb[r] = zeros16
        return 0

    lax.fori_loop(0, RPW, zloop, 0)

    def wpass(c, wbuf, jidx, sem, limit):
        def grp(g, _2):
            base = c * CHUNK + g * 16
            codes = codebuf[pl.ds(base, 16)]
            rv = (codes >> 11) & (RPW - 1)
            jv = codes & (N - 1)
            jidx[pl.ds(g * 16, 16)] = jv
            valid = (jnp.full((16,), base, i32) + iota) < limit
            rows16 = jnp.full((16,), g * 16, i32) + iota
            for h in range(8):
                sh = plsc.load_gather(sbuf, [rv * 8 + h])
                mh = plsc.load_gather(mbuf, [rv * 8 + h])
                nh = plsc.load_gather(nbuf, [jv * 8 + h])
                e = sh + nh
                e = jnp.maximum(e, 0.2 * e)
                w = jnp.exp(e - mh)
                w = jnp.where(valid, w, 0.0)
                plsc.store_scatter(
                    wbuf, [rows16, jnp.full((16,), h, i32)], w)
            return 0

        lax.fori_loop(0, NGRP, grp, 0)

    def rowproc(c, wbuf, jidx, rows_v, sem, limit):
        pltpu.make_async_copy(xp.at[jidx], rows_v, sem).wait()

        cfirst = codebuf[pl.ds(c * CHUNK, 16)]
        rlo = jnp.sum(jnp.where(iota == 0, (cfirst >> 11) & (RPW - 1), 0))
        last = jnp.minimum((c + 1) * CHUNK, limit) - 1
        clast = codebuf[pl.ds((last >> 4) << 4, 16)]
        rhi = jnp.sum(jnp.where(iota == (last & 15),
                                (clast >> 11) & (RPW - 1), 0))

        def row(r, _2):
            lo = jnp.maximum(rp[r] - c * CHUNK, 0)
            hi = jnp.minimum(rp[r + 1] - c * CHUNK, CHUNK)

            @pl.when(hi > lo)
            def _():
                def edge(t, carry):
                    a0, a1, a2, a3, z = carry
                    wrow = wbuf[t]
                    z = z + wrow
                    accs = []
                    for k, ak in enumerate((a0, a1, a2, a3)):
                        xv = rows_v[t, pl.ds(k * 16, 16)]
                        wk = _vgather(wrow, 2 * k + lane3)
                        accs.append(ak + xv * wk)
                    return (*accs, z)

                init = (zeros16,) * 5
                a0, a1, a2, a3, z = lax.fori_loop(lo, hi, edge, init)
                for k, ak in enumerate((a0, a1, a2, a3)):
                    plsc.addupdate(accb.at[r, pl.ds(k * 16, 16)], ak)
                plsc.addupdate(zb.at[r], z)
            return 0

        lax.fori_loop(rlo, rhi + 1, row, 0)

    def fire1(c, limit):
        for u in range(4):
            @pl.when((c & 3) == u)
            def _(u=u):
                wpass(c, wbufs[u], jidxs[u], sem_gs[u], limit)
                pltpu.make_async_copy(
                    xp.at[jidxs[u]], rowss[u], sem_gs[u]).start()

    def proc1(c, limit):
        for u in range(4):
            @pl.when((c & 3) == u)
            def _(u=u):
                rowproc(c, wbufs[u], jidxs[u], rowss[u], sem_gs[u], limit)

    rp[0] = 0
    pltpu.make_async_copy(adj.at[pl.ds(row0, 4)], abuf_a, sem_a).start()

    def block(bi, ptr, rpvec, cf, buf, sem, nxt_buf, nxt_sem):
        pltpu.make_async_copy(
            adj.at[pl.ds(row0 + bi * 4, 4)], buf, sem).wait()

        @pl.when(bi + 1 < NBLK)
        def _():
            pltpu.make_async_copy(
                adj.at[pl.ds(row0 + (bi + 1) * 4, 4)],
                nxt_buf, nxt_sem).start()

        for rr in range(4):
            r = bi * 4 + rr

            def quad(k4, p):
                for u in range(4):
                    k = k4 * 4 + u
                    av = buf[rr, pl.ds(k * 16, 16)]
                    msk = av > 0.0
                    cntv = plsc.all_reduce_population_count(msk)
                    cvec = jnp.full((16,), r * N + k * 16, i32) + iota
                    pc = jnp.minimum(p, ECAP - 16)
                    plsc.store_compressed(codebuf.at[pl.ds(pc, 16)], cvec,
                                          mask=msk)
                    p = p + cntv[0]
                return p

            ptr = lax.fori_loop(0, N // 64, quad, ptr)
            pmin = jnp.minimum(ptr, ECAP)
            rp[r + 1] = pmin
            iloc = (r + 1) & 15
            rpvec = jnp.where(iota == iloc, jnp.full((16,), pmin, i32),
                              rpvec)

            @pl.when(iloc == 15)
            def _():
                rpbuf[pl.ds(r + 1 - 15, 16)] = rpvec

            rpvec = jnp.where(jnp.full((16,), iloc == 15, jnp.bool_),
                              izeros16, rpvec)

        return ptr, rpvec, cf

    def block2(ob, carry):
        ptr, rpvec, cf = carry
        ptr, rpvec, cf = block(2 * ob, ptr, rpvec, cf,
                               abuf_a, sem_a, abuf_b, sem_b)
        ptr, rpvec, cf = block(2 * ob + 1, ptr, rpvec, cf,
                               abuf_b, sem_b, abuf_a, sem_a)
        return ptr, rpvec, cf

    total, rpvec, cf = lax.fori_loop(0, NBLK // 2, block2,
                                     (0, izeros16, 0))
    total = jnp.minimum(total, ECAP)
    rpbuf[pl.ds(RPW, 16)] = rpvec

    pltpu.sync_copy(codebuf, codes_out.at[wid])
    pltpu.sync_copy(rpbuf, rp_out.at[wid])

    nact = (total + CHUNK - 1) // CHUNK

    def dfire(c, _):
        fire1(c, total)

        @pl.when(c >= 3)
        def _():
            proc1(c - 3, total)
        return 0

    lax.fori_loop(cf, nact, dfire, 0)
    lax.fori_loop(jnp.maximum(nact - 3, 0), nact,
                  lambda c, _: (proc1(c, total), 0)[1], 0)

    def fin(r, _):
        zrow = zb[r]
        for k in range(4):
            zk = _vgather(zrow, 2 * k + lane3)
            t = accb[r, pl.ds(k * 16, 16)] / zk + bbuf[pl.ds(k * 16, 16)]
            outb[r, pl.ds(k * 16, 16)] = jnp.where(t > 0.0, t,
                                                   jnp.exp(t) - 1.0)
        return 0

    lax.fori_loop(0, RPW, fin, 0)
    pltpu.sync_copy(outb, out.at[pl.ds(row0, RPW)])


@functools.partial(
    pl.kernel,
    out_type=jax.ShapeDtypeStruct((SCROWS, 16), jnp.float32),
    mesh=_MESH,
    compiler_params=_SC_PARAMS,
    scratch_types=[
        pltpu.VMEM((ECAP,), jnp.int32),
        pltpu.VMEM((N * 16,), jnp.float32),
        pltpu.VMEM((N,), jnp.float32),
        pltpu.VMEM((RPW,), jnp.float32),
        pltpu.VMEM((RPW,), jnp.float32),
        pltpu.VMEM((ECAP,), jnp.float32),
        pltpu.VMEM((RPW, 16), jnp.float32),
        pltpu.VMEM((RPW, 16), jnp.float32),
        pltpu.VMEM((RPW, 16), jnp.float32),
        pltpu.VMEM((16,), jnp.float32),
        pltpu.VMEM((80,), jnp.int32),
        pltpu.SMEM((72,), jnp.int32),
    ],
)
def _sc_layer2(xp2flat, s2, n2, m2, bias2, codes_in, rp_in,
               out,
               codebuf, xbuf, nbuf, sbuf, mbuf, wb,
               accb, zb, outb, bbuf, rpbuf, rp):
    wid = _wid()
    row0 = wid * RPW
    iota = _iota()
    f32 = jnp.float32
    i32 = jnp.int32
    zeros16 = jnp.zeros((16,), f32)

    pltpu.sync_copy(codes_in.at[wid], codebuf)
    pltpu.sync_copy(rp_in.at[wid], rpbuf)
    pltpu.sync_copy(xp2flat, xbuf)
    pltpu.sync_copy(n2, nbuf)
    pltpu.sync_copy(s2.at[pl.ds(row0, RPW)], sbuf)
    pltpu.sync_copy(m2.at[pl.ds(row0, RPW)], mbuf)
    pltpu.sync_copy(bias2, bbuf)

    def rpl(i, _):
        vec = rpbuf[pl.ds((i >> 4) << 4, 16)]
        rp[i] = jnp.sum(jnp.where(iota == (i & 15), vec, 0))
        return 0

    lax.fori_loop(0, RPW + 1, rpl, 0)
    total = rp[RPW]

    def grp(g, _):
        base = g * 16
        codes = codebuf[pl.ds(base, 16)]
        rv = (codes >> 11) & (RPW - 1)
        jv = codes & (N - 1)
        valid = (jnp.full((16,), base, i32) + iota) < total
        sh = plsc.load_gather(sbuf, [rv])
        mh = plsc.load_gather(mbuf, [rv])
        nh = plsc.load_gather(nbuf, [jv])
        e = sh + nh
        e = jnp.maximum(e, 0.2 * e)
        w = jnp.exp(e - mh)
        wb[pl.ds(base, 16)] = jnp.where(valid, w, 0.0)
        return 0

    ngrp_used = (total + 15) >> 4
    lax.fori_loop(0, ngrp_used, grp, 0)

    def row(r, _):
        lo = rp[r]
        hi = rp[r + 1]

        @pl.when(hi > lo)
        def _():
            def edge(t, carry):
                av, z = carry
                tb = (t >> 4) << 4
                tl = t & 15
                w16 = wb[pl.ds(tb, 16)]
                wsp = _vgather(w16, jnp.full((16,), tl, i32))
                jvec = codebuf[pl.ds(tb, 16)]
                j = jnp.sum(jnp.where(iota == tl, jvec & (N - 1), 0))
                xv = xbuf[pl.ds(j * 16, 16)]
                return (av + xv * wsp, z + wsp)

            av, z = lax.fori_loop(lo, hi, edge, (zeros16, zeros16))
            accb[r] = av
            zb[r] = z
        return 0

    def zloop(r, _):
        accb[r] = zeros16
        zb[r] = jnp.full((16,), 1.0, f32)
        return 0

    lax.fori_loop(0, RPW, zloop, 0)
    lax.fori_loop(0, RPW, row, 0)

    neg = jnp.full((16,), -1e30, f32)

    def fin(r, _):
        t = accb[r] / zb[r] + bbuf[...]
        t = jnp.where(iota < 3, t, neg)
        ex = jnp.exp(t - jnp.max(t))
        outb[r] = ex / jnp.sum(ex)
        return 0

    lax.fori_loop(0, RPW, fin, 0)
    pltpu.sync_copy(outb, out.at[pl.ds(row0, RPW)])


def _agg_body(a_ref, s_ref, m_ref, n_ref, xp_ref, b_ref, out_ref, acc_ref,
              z_ref, *, heads, chan, n_j_blocks, act):
    j = pl.program_id(1)

    @pl.when(j == 0)
    def _():
        acc_ref[...] = jnp.zeros_like(acc_ref)
        z_ref[...] = jnp.zeros_like(z_ref)

    a_blk = a_ref[...]
    for h in range(heads):
        e = s_ref[:, h:h + 1] + n_ref[:, h].reshape(1, -1)
        e = jnp.maximum(e, 0.2 * e)
        p = a_blk * jnp.exp(e - m_ref[:, h:h + 1])
        z_ref[:, h:h + 1] += jnp.sum(p, axis=1, keepdims=True)
        acc_ref[:, h * chan:(h + 1) * chan] += jnp.dot(
            p, xp_ref[:, h * chan:(h + 1) * chan],
            preferred_element_type=jnp.float32)

    @pl.when(j == n_j_blocks - 1)
    def _():
        acc = acc_ref[...]
        parts = [acc[:, h * chan:(h + 1) * chan] / z_ref[:, h:h + 1]
                 for h in range(heads)]
        t = jnp.concatenate(parts, axis=1) if len(parts) > 1 else parts[0]
        t = t + b_ref[...]
        out_ref[...] = act(t)


def _elu(t):
    return jnp.where(t > 0, t, jnp.exp(t) - 1.0)


def _row_softmax(t):
    m = jnp.max(t, axis=1, keepdims=True)
    ex = jnp.exp(t - m)
    return ex / jnp.sum(ex, axis=1, keepdims=True)


def _tc_agg(adj, xp, s, n, m, brow, heads, chan, act, rb=256, cb=512):
    n_nodes = adj.shape[0]
    hc = xp.shape[1]
    out_c = heads * chan
    i_off = SCROWS // rb
    grid = ((n_nodes - SCROWS) // rb, n_nodes // cb)
    body = functools.partial(_agg_body, heads=heads, chan=chan,
                             n_j_blocks=grid[1], act=act)
    return pl.pallas_call(
        body,
        grid=grid,
        in_specs=[
            pl.BlockSpec((rb, cb), lambda i, j: (i + i_off, j)),
            pl.BlockSpec((rb, heads), lambda i, j: (i + i_off, 0)),
            pl.BlockSpec((rb, heads), lambda i, j: (i + i_off, 0)),
            pl.BlockSpec((cb, heads), lambda i, j: (j, 0)),
            pl.BlockSpec((cb, hc), lambda i, j: (j, 0)),
            pl.BlockSpec((1, out_c), lambda i, j: (0, 0)),
        ],
        out_specs=pl.BlockSpec((rb, out_c), lambda i, j: (i, 0)),
        out_shape=jax.ShapeDtypeStruct((n_nodes - SCROWS, out_c),
                                       jnp.float32),
        scratch_shapes=[
            pltpu.VMEM((rb, out_c), jnp.float32),
            pltpu.VMEM((rb, heads), jnp.float32),
        ],
        compiler_params=pltpu.CompilerParams(
            dimension_semantics=("parallel", "arbitrary")),
    )(adj, s, m, n, xp, brow)


def kernel(x, fltr, W1, a1_self, a1_neigh, b1, W2, a2_self, a2_neigh, b2):
    f, h1, c1 = W1.shape
    hc1, h2, c2 = W2.shape
    f32 = jnp.float32

    w1pad = jnp.zeros((f, 128), f32).at[:, :h1 * c1].set(
        W1.reshape(f, h1 * c1))
    a1s = jnp.zeros((128, h1), f32).at[:h1 * c1].set(_head_proj(a1_self, h1))
    a1n = jnp.zeros((128, h1), f32).at[:h1 * c1].set(_head_proj(a1_neigh, h1))
    xp1, s1, n1, m1 = _prep(x, w1pad, a1s, a1n)

    hid_sc, codes, rps = _sc_layer1(
        fltr, xp1, s1.reshape(-1), n1.reshape(-1), m1.reshape(-1), b1)
    hid_tc = _tc_agg(fltr, xp1, s1, n1, m1, b1.reshape(1, -1),
                     h1, c1, _elu)
    hid = jnp.concatenate([hid_sc, hid_tc], axis=0)

    w2pad = jnp.zeros((hc1, 16), f32).at[:, :c2].set(W2.reshape(hc1, h2 * c2))
    a2s = jnp.zeros((16, 1), f32).at[:c2].set(_head_proj(a2_self, h2))
    a2n = jnp.zeros((16, 1), f32).at[:c2].set(_head_proj(a2_neigh, h2))
    b2pad = jnp.zeros((16,), f32).at[:c2].set(b2)

    xp2, s2, n2, m2 = _prep(hid, w2pad, a2s, a2n)
    out_sc = _sc_layer2(xp2.reshape(-1), s2.reshape(-1), n2.reshape(-1),
                        m2.reshape(-1), b2pad, codes, rps)
    out_tc = _tc_agg(fltr, xp2, s2, n2, m2, b2pad[:c2].reshape(1, -1),
                     h2, c2, _row_softmax)
    return jnp.concatenate([out_sc[:, :c2], out_tc], axis=0)
